# baseline (speedup 1.0000x reference)
import functools

import jax
import jax.numpy as jnp
from jax import lax
from jax.experimental import pallas as pl
from jax.experimental.pallas import tpu as pltpu
from jax.experimental.pallas import tpu_sc as plsc

LANES = 16
CHUNK = 40


def _pos_table(seq_len, dim):
    even_i = jnp.arange(0, dim, 2).astype(jnp.float32)
    denom = jnp.power(10000.0, even_i / dim)
    position = jnp.arange(seq_len, dtype=jnp.float32).reshape(seq_len, 1)
    even_pe = jnp.sin(position / denom)
    odd_pe = jnp.cos(position / denom)
    return jnp.stack([even_pe, odd_pe], axis=2).reshape(seq_len, dim)


@functools.partial(jax.jit, static_argnames=("n_rows", "seq", "dim", "nc", "ns"))
def _sc_embed(idx, emb, pe, n_rows, seq, dim, nc, ns):
    nw = nc * ns
    rows_per_w = n_rows // nw
    seqs_per_w = rows_per_w // seq
    mesh = plsc.VectorSubcoreMesh(core_axis_name="c", subcore_axis_name="s")

    @functools.partial(
        pl.kernel,
        mesh=mesh,
        out_type=jax.ShapeDtypeStruct((n_rows, dim), jnp.float32),
        scratch_types=[
            pltpu.VMEM((seq, dim), jnp.float32),
            pltpu.VMEM((seq,), jnp.int32),
            pltpu.VMEM((seq, dim), jnp.float32),
            pltpu.SemaphoreType.DMA,
        ],
        compiler_params=pltpu.CompilerParams(use_tc_tiling_on_sc=False),
    )
    def k(idx_hbm, emb_hbm, pe_hbm, out_hbm, pe_v, idx_v, rows_v, gsem):
        wid = lax.axis_index("s") * nc + lax.axis_index("c")
        base = wid * rows_per_w
        pltpu.sync_copy(pe_hbm, pe_v)

        def seq_body(s, carry):
            row0 = base + s * seq
            pltpu.sync_copy(idx_hbm.at[pl.ds(row0, seq)], idx_v)
            copies = []
            for c in range(seq // CHUNK):
                copies.append(
                    pltpu.async_copy(
                        emb_hbm.at[idx_v.at[pl.ds(c * CHUNK, CHUNK)]],
                        rows_v.at[pl.ds(c * CHUNK, CHUNK)],
                        gsem,
                    )
                )
            for cp in copies:
                cp.wait()

            def row_body(i, c2):
                for j in range(dim // LANES):
                    plsc.addupdate(
                        rows_v.at[i, pl.ds(j * LANES, LANES)],
                        pe_v[i, pl.ds(j * LANES, LANES)],
                    )
                return c2

            lax.fori_loop(0, seq, row_body, 0)
            pltpu.sync_copy(rows_v, out_hbm.at[pl.ds(row0, seq)])
            return carry

        lax.fori_loop(0, seqs_per_w, seq_body, 0)

    return k(idx, emb, pe)


def kernel(x, embedding):
    b, s = x.shape
    v, d = embedding.shape
    idx = x.reshape(-1).astype(jnp.int32)
    pe = _pos_table(s, d)
    info = plsc.get_sparse_core_info()
    out = _sc_embed(idx, embedding, pe, b * s, s, d,
                    info.num_cores, info.num_subcores)
    return out.reshape(b, s, d)


# speedup vs baseline: 1.1708x; 1.1708x over previous
import functools

import jax
import jax.numpy as jnp
from jax import lax
from jax.experimental import pallas as pl
from jax.experimental.pallas import tpu as pltpu
from jax.experimental.pallas import tpu_sc as plsc

LANES = 16
CHUNK = 40
GROUP = 2
ROW_UNROLL = 4


def _pos_table(seq_len, dim):
    even_i = jnp.arange(0, dim, 2).astype(jnp.float32)
    denom = jnp.power(10000.0, even_i / dim)
    position = jnp.arange(seq_len, dtype=jnp.float32).reshape(seq_len, 1)
    even_pe = jnp.sin(position / denom)
    odd_pe = jnp.cos(position / denom)
    return jnp.stack([even_pe, odd_pe], axis=2).reshape(seq_len, dim)


@functools.partial(jax.jit, static_argnames=("n_rows", "seq", "dim", "nc", "ns"))
def _sc_embed(idx, emb, pe, n_rows, seq, dim, nc, ns):
    nw = nc * ns
    rows_per_w = n_rows // nw
    seqs_per_w = rows_per_w // seq
    n_groups = seqs_per_w // GROUP
    grows = GROUP * seq
    mesh = plsc.VectorSubcoreMesh(core_axis_name="c", subcore_axis_name="s")

    @functools.partial(
        pl.kernel,
        mesh=mesh,
        out_type=jax.ShapeDtypeStruct((n_rows, dim), jnp.float32),
        scratch_types=[
            pltpu.VMEM((grows, dim), jnp.float32),
            pltpu.VMEM((grows,), jnp.int32),
            pltpu.VMEM((grows,), jnp.int32),
            pltpu.VMEM((grows, dim), jnp.float32),
            pltpu.VMEM((grows, dim), jnp.float32),
            pltpu.SemaphoreType.DMA,
            pltpu.SemaphoreType.DMA,
            pltpu.SemaphoreType.DMA,
            pltpu.SemaphoreType.DMA,
            pltpu.SemaphoreType.DMA,
            pltpu.SemaphoreType.DMA,
        ],
        compiler_params=pltpu.CompilerParams(use_tc_tiling_on_sc=False),
    )
    def k(idx_hbm, emb_hbm, pe_hbm, out_hbm,
          pe_v, idx0, idx1, rows0, rows1, g0, g1, i0, i1, o0, o1):
        idx_v = (idx0, idx1)
        rows_v = (rows0, rows1)
        gsem = (g0, g1)
        isem = (i0, i1)
        osem = (o0, o1)
        wid = lax.axis_index("s") * nc + lax.axis_index("c")
        base = wid * rows_per_w
        pltpu.sync_copy(pe_hbm, pe_v)

        def fire_gathers(b, g):
            for c in range(grows // CHUNK):
                pltpu.async_copy(
                    emb_hbm.at[idx_v[b].at[pl.ds(c * CHUNK, CHUNK)]],
                    rows_v[b].at[pl.ds(c * CHUNK, CHUNK)],
                    gsem[b],
                )

        def fire_idx(b, g):
            row0 = base + g * grows
            pltpu.async_copy(idx_hbm.at[pl.ds(row0, grows)], idx_v[b], isem[b])

        pltpu.sync_copy(idx_hbm.at[pl.ds(base, grows)], idx_v[0])
        fire_gathers(0, 0)
        fire_idx(1, 1)

        def body(gg, carry):
            for b in range(2):
                g = 2 * gg + b
                pltpu.make_async_copy(
                    out_hbm.at[pl.ds(base, grows)], rows_v[b], gsem[b]).wait()

                @pl.when(g + 2 < n_groups)
                def _():
                    fire_idx(b, g + 2)

                @pl.when(g >= 1)
                def _():
                    pltpu.make_async_copy(
                        rows_v[1 - b], out_hbm.at[pl.ds(base, grows)],
                        osem[1 - b]).wait()

                @pl.when(g + 1 < n_groups)
                def _():
                    pltpu.make_async_copy(
                        idx_hbm.at[pl.ds(base, grows)], idx_v[1 - b],
                        isem[1 - b]).wait()
                    fire_gathers(1 - b, g + 1)

                def row_body(i, c2):
                    for u in range(ROW_UNROLL):
                        for j in range(dim // LANES):
                            plsc.addupdate(
                                rows_v[b].at[i + u, pl.ds(j * LANES, LANES)],
                                pe_v[i + u, pl.ds(j * LANES, LANES)],
                            )
                    return c2

                lax.fori_loop(0, grows // ROW_UNROLL,
                              lambda i, c: row_body(i * ROW_UNROLL, c), 0)

                pltpu.async_copy(
                    rows_v[b], out_hbm.at[pl.ds(base + g * grows, grows)],
                    osem[b])
            return carry

        lax.fori_loop(0, n_groups // 2, body, 0)

        lb = (n_groups - 1) % 2
        pltpu.make_async_copy(
            rows_v[lb], out_hbm.at[pl.ds(base, grows)], osem[lb]).wait()

    return k(idx, emb, pe)


def kernel(x, embedding):
    b, s = x.shape
    v, d = embedding.shape
    idx = x.reshape(-1).astype(jnp.int32)
    pe = jnp.tile(_pos_table(s, d), (GROUP, 1))
    info = plsc.get_sparse_core_info()
    out = _sc_embed(idx, embedding, pe, b * s, s, d,
                    info.num_cores, info.num_subcores)
    return out.reshape(b, s, d)


# speedup vs baseline: 1.4558x; 1.2434x over previous
import functools

import jax
import jax.numpy as jnp
from jax import lax
from jax.experimental import pallas as pl
from jax.experimental.pallas import tpu as pltpu
from jax.experimental.pallas import tpu_sc as plsc

LANES = 16
CHUNK = 40
GROUP = 1
ROW_UNROLL = 4
PAD_DIM = 128


def _pos_table(seq_len, dim):
    even_i = jnp.arange(0, dim, 2).astype(jnp.float32)
    denom = jnp.power(10000.0, even_i / dim)
    position = jnp.arange(seq_len, dtype=jnp.float32).reshape(seq_len, 1)
    even_pe = jnp.sin(position / denom)
    odd_pe = jnp.cos(position / denom)
    return jnp.stack([even_pe, odd_pe], axis=2).reshape(seq_len, dim)


@functools.partial(jax.jit, static_argnames=("n_rows", "seq", "dim", "nc", "ns"))
def _sc_embed(idx, emb, pe, n_rows, seq, dim, nc, ns):
    nw = nc * ns
    rows_per_w = n_rows // nw
    seqs_per_w = rows_per_w // seq
    n_groups = seqs_per_w // GROUP
    grows = GROUP * seq
    mesh = plsc.VectorSubcoreMesh(core_axis_name="c", subcore_axis_name="s")

    @functools.partial(
        pl.kernel,
        mesh=mesh,
        out_type=jax.ShapeDtypeStruct((n_rows, PAD_DIM), jnp.float32),
        scratch_types=[
            pltpu.VMEM((grows, dim), jnp.float32),
            pltpu.VMEM((grows,), jnp.int32),
            pltpu.VMEM((grows,), jnp.int32),
            pltpu.VMEM((grows, PAD_DIM), jnp.float32),
            pltpu.VMEM((grows, PAD_DIM), jnp.float32),
            pltpu.SemaphoreType.DMA,
            pltpu.SemaphoreType.DMA,
            pltpu.SemaphoreType.DMA,
            pltpu.SemaphoreType.DMA,
            pltpu.SemaphoreType.DMA,
            pltpu.SemaphoreType.DMA,
        ],
        compiler_params=pltpu.CompilerParams(use_tc_tiling_on_sc=True),
    )
    def k(idx_hbm, emb_hbm, pe_hbm, out_hbm,
          pe_v, idx0, idx1, rows0, rows1, g0, g1, i0, i1, o0, o1):
        idx_v = (idx0, idx1)
        rows_v = (rows0, rows1)
        gsem = (g0, g1)
        isem = (i0, i1)
        osem = (o0, o1)
        wid = lax.axis_index("s") * nc + lax.axis_index("c")
        base = wid * rows_per_w
        pltpu.sync_copy(pe_hbm, pe_v)

        def out_slice(g):
            return out_hbm.at[pl.ds(base + g * grows, grows)]

        def fire_gathers(b, g):
            for c in range(grows // CHUNK):
                pltpu.async_copy(
                    emb_hbm.at[idx_v[b].at[pl.ds(c * CHUNK, CHUNK)]],
                    rows_v[b].at[pl.ds(c * CHUNK, CHUNK)],
                    gsem[b],
                )

        def fire_idx(b, g):
            row0 = base + g * grows
            pltpu.async_copy(idx_hbm.at[pl.ds(row0, grows)], idx_v[b], isem[b])

        pltpu.sync_copy(idx_hbm.at[pl.ds(base, grows)], idx_v[0])
        fire_gathers(0, 0)
        fire_idx(1, 1)

        def body(gg, carry):
            for b in range(2):
                g = 2 * gg + b
                pltpu.make_async_copy(
                    emb_hbm.at[pl.ds(0, grows)], rows_v[b], gsem[b]).wait()

                @pl.when(g + 2 < n_groups)
                def _():
                    fire_idx(b, g + 2)

                @pl.when(g >= 1)
                def _():
                    pltpu.make_async_copy(
                        rows_v[1 - b], out_slice(0), osem[1 - b]).wait()

                @pl.when(g + 1 < n_groups)
                def _():
                    pltpu.make_async_copy(
                        idx_hbm.at[pl.ds(base, grows)], idx_v[1 - b],
                        isem[1 - b]).wait()
                    fire_gathers(1 - b, g + 1)

                def row_body(i, c2):
                    for u in range(ROW_UNROLL):
                        for j in range(dim // LANES):
                            plsc.addupdate(
                                rows_v[b].at[i + u, pl.ds(j * LANES, LANES)],
                                pe_v[i + u, pl.ds(j * LANES, LANES)],
                            )
                    return c2

                lax.fori_loop(0, grows // ROW_UNROLL,
                              lambda i, c: row_body(i * ROW_UNROLL, c), 0)

                pltpu.async_copy(rows_v[b], out_slice(g), osem[b])
            return carry

        lax.fori_loop(0, n_groups // 2, body, 0)

        lb = (n_groups - 1) % 2
        pltpu.make_async_copy(rows_v[lb], out_slice(0), osem[lb]).wait()

    return k(idx, emb, pe)


def kernel(x, embedding):
    b, s = x.shape
    v, d = embedding.shape
    idx = x.reshape(-1).astype(jnp.int32)
    pe = jnp.tile(_pos_table(s, d), (GROUP, 1))
    emb128 = jnp.pad(embedding, ((0, 0), (0, PAD_DIM - d)))
    info = plsc.get_sparse_core_info()
    out = _sc_embed(idx, emb128, pe, b * s, s, d,
                    info.num_cores, info.num_subcores)
    return out[:, :d].reshape(b, s, d)
